# baseline (speedup 1.0000x reference)
import functools

import jax
import jax.numpy as jnp
from jax import lax
from jax.experimental import pallas as pl
from jax.experimental.pallas import tpu as pltpu
from jax.experimental.pallas import tpu_sc as plsc

N, C, H, W = 64, 384, 28, 28
R = H * W * N
LANES = 16
C_CHUNKS = C // LANES
NW = 32
ROWS_PER_WORKER = R // NW
BIG = 112
NBIG = ROWS_PER_WORKER // BIG
SMALL = 56
PAIRS = ROWS_PER_WORKER // (2 * SMALL)


def _body(x_hbm, mask_hbm, out_hbm, mask_v, sel_v, bufs_v,
          gsem0, gsem1, gsem2, ssem0, ssem1, ssem2):
    info = plsc.get_sparse_core_info()
    wid = lax.axis_index("s") * info.num_cores + lax.axis_index("c")
    base = wid * ROWS_PER_WORKER

    gsems = (gsem0, gsem1, gsem2)
    ssems = (ssem0, ssem1, ssem2)
    inb = (bufs_v.at[0, pl.ds(0, SMALL)], bufs_v.at[0, pl.ds(SMALL, SMALL)])
    outb = (bufs_v.at[1, pl.ds(0, SMALL)], bufs_v.at[1, pl.ds(SMALL, SMALL)])

    def small_src(c):
        return x_hbm.at[pl.ds(base + c * SMALL, SMALL)]

    def small_dst(c):
        return out_hbm.at[pl.ds(base + c * SMALL, SMALL)]

    def big_src(c):
        return x_hbm.at[pl.ds(base + c * BIG, BIG)]

    def big_dst(c):
        return out_hbm.at[pl.ds(base + c * BIG, BIG)]

    pltpu.async_copy(small_src(0), inb[0], gsem0)
    pltpu.async_copy(small_src(1), inb[1], gsem0)

    pltpu.sync_copy(mask_hbm, mask_v)

    zero16 = jnp.zeros((LANES,), jnp.int32)
    zero16f = jnp.zeros((LANES,), jnp.float32)
    one16 = jnp.ones((LANES,), jnp.int32)
    iota16 = lax.iota(jnp.int32, LANES)
    for t in range(C_CHUNKS):
        sel_v[pl.ds(t * LANES, LANES)] = zero16
    offset = jnp.int32(0)
    for t in range(C_CHUNKS):
        xv = mask_v[pl.ds(t * LANES, LANES)]
        m = xv != zero16f
        mi = m.astype(jnp.int32)
        pos = plsc.cumsum(mi)
        ids = iota16 + jnp.full((LANES,), t * LANES, jnp.int32)
        offv = lax.broadcast_in_dim(offset, (LANES,), ())
        plsc.store_scatter(sel_v, [pos + offv - one16], ids, mask=m)
        offset = offset + jnp.sum(mi)

    ident = jnp.bool_(True)
    for t in range(C_CHUNKS):
        sv = sel_v[pl.ds(t * LANES, LANES)]
        ids = iota16 + jnp.full((LANES,), t * LANES, jnp.int32)
        ident = jnp.logical_and(ident, jnp.all(sv == ids))

    ident = jnp.logical_and(ident, jnp.bool_(False))

    @pl.when(ident)
    def _copy_mode():
        pltpu.async_copy(big_src(1), bufs_v.at[1], gsems[1])
        pltpu.async_copy(big_src(2), bufs_v.at[2], gsems[2])
        for c in range(NBIG):
            k = c % 3
            pltpu.make_async_copy(big_src(0), bufs_v.at[k], gsems[k]).wait()
            pltpu.async_copy(bufs_v.at[k], big_dst(c), ssems[k])
            if c >= 2 and c + 1 < NBIG:
                kn = (c + 1) % 3
                pltpu.make_async_copy(
                    bufs_v.at[kn], big_dst(0), ssems[kn]).wait()
                pltpu.async_copy(big_src(c + 1), bufs_v.at[kn], gsems[kn])
        for c in range(NBIG - 2, NBIG):
            pltpu.make_async_copy(
                bufs_v.at[c % 3], big_dst(0), ssems[c % 3]).wait()

    @pl.when(jnp.logical_not(ident))
    def _gather_mode():
        sel_regs = tuple(
            sel_v[pl.ds(t * LANES, LANES)] for t in range(C_CHUNKS))

        def compute(slot, sels):
            src = inb[slot]
            dst = outb[slot]

            def row(r, sels):
                rv = lax.broadcast_in_dim(r, (LANES,), ())
                for t in range(C_CHUNKS):
                    v = plsc.load_gather(src, [rv, sels[t]])
                    dst[r, pl.ds(t * LANES, LANES)] = v
                return sels

            return lax.fori_loop(0, SMALL, row, sels)

        def pair(j, sels):
            for slot in range(2):
                c = 2 * j + slot
                if slot == 0:
                    @pl.when(j == 0)
                    def _():
                        pltpu.make_async_copy(
                            big_src(0), bufs_v.at[0], gsem0).wait()

                    @pl.when(j > 0)
                    def _():
                        pltpu.make_async_copy(
                            small_src(0), inb[0], gsem0).wait()
                else:
                    @pl.when(j > 0)
                    def _():
                        pltpu.make_async_copy(
                            small_src(0), inb[1], gsems[1]).wait()

                @pl.when(j > 0)
                def _():
                    pltpu.make_async_copy(
                        outb[slot], small_dst(0), ssems[slot]).wait()
                sels = compute(slot, sels)
                pltpu.async_copy(outb[slot], small_dst(c), ssems[slot])

                @pl.when(j < PAIRS - 1)
                def _():
                    pltpu.async_copy(small_src(c + 2), inb[slot],
                                     gsems[slot])
            return sels

        lax.fori_loop(0, PAIRS, pair, sel_regs)
        for slot in range(2):
            pltpu.make_async_copy(
                outb[slot], small_dst(0), ssems[slot]).wait()


def kernel(input_tensor, indexes):
    x = input_tensor.transpose(2, 3, 0, 1).reshape(R, C)
    mesh = plsc.VectorSubcoreMesh(core_axis_name="c", subcore_axis_name="s")
    run = functools.partial(
        pl.kernel,
        mesh=mesh,
        compiler_params=pltpu.CompilerParams(
            use_tc_tiling_on_sc=True, needs_layout_passes=False),
        out_type=jax.ShapeDtypeStruct((R, C), jnp.float32),
        scratch_types=[
            pltpu.VMEM((C,), jnp.float32),
            pltpu.VMEM((C,), jnp.int32),
            pltpu.VMEM((3, BIG, C), jnp.float32),
            pltpu.SemaphoreType.DMA,
            pltpu.SemaphoreType.DMA,
            pltpu.SemaphoreType.DMA,
            pltpu.SemaphoreType.DMA,
            pltpu.SemaphoreType.DMA,
            pltpu.SemaphoreType.DMA,
        ],
    )(_body)
    out = run(x, indexes)
    return out.reshape(H, W, N, C).transpose(2, 3, 0, 1)


# speedup vs baseline: 2.3302x; 2.3302x over previous
import functools

import jax
import jax.numpy as jnp
from jax import lax
from jax.experimental import pallas as pl
from jax.experimental.pallas import tpu as pltpu
from jax.experimental.pallas import tpu_sc as plsc

N, C, H, W = 64, 384, 28, 28
R = H * W * N
LANES = 16
C_CHUNKS = C // LANES
NW = 32
ROWS_PER_WORKER = R // NW
BIG = 112
NBIG = ROWS_PER_WORKER // BIG
SMALL = 56
PAIRS = ROWS_PER_WORKER // (2 * SMALL)


def _body(x_hbm, mask_hbm, out_hbm, mask_v, sel_v, bufs_v,
          gsem0, gsem1, gsem2, ssem0, ssem1, ssem2):
    info = plsc.get_sparse_core_info()
    wid = lax.axis_index("s") * info.num_cores + lax.axis_index("c")
    base = wid * ROWS_PER_WORKER

    gsems = (gsem0, gsem1, gsem2)
    ssems = (ssem0, ssem1, ssem2)
    inb = (bufs_v.at[0, pl.ds(0, SMALL)], bufs_v.at[0, pl.ds(SMALL, SMALL)])
    outb = (bufs_v.at[1, pl.ds(0, SMALL)], bufs_v.at[1, pl.ds(SMALL, SMALL)])

    def small_src(c):
        return x_hbm.at[pl.ds(base + c * SMALL, SMALL)]

    def small_dst(c):
        return out_hbm.at[pl.ds(base + c * SMALL, SMALL)]

    def big_src(c):
        return x_hbm.at[pl.ds(base + c * BIG, BIG)]

    def big_dst(c):
        return out_hbm.at[pl.ds(base + c * BIG, BIG)]

    pltpu.async_copy(small_src(0), inb[0], gsem0)
    pltpu.async_copy(small_src(1), inb[1], gsem0)

    pltpu.sync_copy(mask_hbm, mask_v)

    zero16 = jnp.zeros((LANES,), jnp.int32)
    zero16f = jnp.zeros((LANES,), jnp.float32)
    one16 = jnp.ones((LANES,), jnp.int32)
    iota16 = lax.iota(jnp.int32, LANES)
    for t in range(C_CHUNKS):
        sel_v[pl.ds(t * LANES, LANES)] = zero16
    offset = jnp.int32(0)
    for t in range(C_CHUNKS):
        xv = mask_v[pl.ds(t * LANES, LANES)]
        m = xv != zero16f
        mi = m.astype(jnp.int32)
        pos = plsc.cumsum(mi)
        ids = iota16 + jnp.full((LANES,), t * LANES, jnp.int32)
        offv = lax.broadcast_in_dim(offset, (LANES,), ())
        plsc.store_scatter(sel_v, [pos + offv - one16], ids, mask=m)
        offset = offset + jnp.sum(mi)

    ident = jnp.bool_(True)
    for t in range(C_CHUNKS):
        sv = sel_v[pl.ds(t * LANES, LANES)]
        ids = iota16 + jnp.full((LANES,), t * LANES, jnp.int32)
        ident = jnp.logical_and(ident, jnp.all(sv == ids))

    @pl.when(ident)
    def _copy_mode():
        pltpu.async_copy(big_src(1), bufs_v.at[1], gsems[1])
        pltpu.async_copy(big_src(2), bufs_v.at[2], gsems[2])
        for c in range(NBIG):
            k = c % 3
            pltpu.make_async_copy(big_src(0), bufs_v.at[k], gsems[k]).wait()
            pltpu.async_copy(bufs_v.at[k], big_dst(c), ssems[k])
            if c >= 2 and c + 1 < NBIG:
                kn = (c + 1) % 3
                pltpu.make_async_copy(
                    bufs_v.at[kn], big_dst(0), ssems[kn]).wait()
                pltpu.async_copy(big_src(c + 1), bufs_v.at[kn], gsems[kn])
        for c in range(NBIG - 2, NBIG):
            pltpu.make_async_copy(
                bufs_v.at[c % 3], big_dst(0), ssems[c % 3]).wait()

    @pl.when(jnp.logical_not(ident))
    def _gather_mode():
        sel_regs = tuple(
            sel_v[pl.ds(t * LANES, LANES)] for t in range(C_CHUNKS))

        def compute(slot, sels):
            src = inb[slot]
            dst = outb[slot]

            def row(r, sels):
                rv = lax.broadcast_in_dim(r, (LANES,), ())
                for t in range(C_CHUNKS):
                    v = plsc.load_gather(src, [rv, sels[t]])
                    dst[r, pl.ds(t * LANES, LANES)] = v
                return sels

            return lax.fori_loop(0, SMALL, row, sels)

        def pair(j, sels):
            for slot in range(2):
                c = 2 * j + slot
                if slot == 0:
                    @pl.when(j == 0)
                    def _():
                        pltpu.make_async_copy(
                            big_src(0), bufs_v.at[0], gsem0).wait()

                    @pl.when(j > 0)
                    def _():
                        pltpu.make_async_copy(
                            small_src(0), inb[0], gsem0).wait()
                else:
                    @pl.when(j > 0)
                    def _():
                        pltpu.make_async_copy(
                            small_src(0), inb[1], gsems[1]).wait()

                @pl.when(j > 0)
                def _():
                    pltpu.make_async_copy(
                        outb[slot], small_dst(0), ssems[slot]).wait()
                sels = compute(slot, sels)
                pltpu.async_copy(outb[slot], small_dst(c), ssems[slot])

                @pl.when(j < PAIRS - 1)
                def _():
                    pltpu.async_copy(small_src(c + 2), inb[slot],
                                     gsems[slot])
            return sels

        lax.fori_loop(0, PAIRS, pair, sel_regs)
        for slot in range(2):
            pltpu.make_async_copy(
                outb[slot], small_dst(0), ssems[slot]).wait()


def kernel(input_tensor, indexes):
    x = input_tensor.transpose(2, 3, 0, 1).reshape(R, C)
    mesh = plsc.VectorSubcoreMesh(core_axis_name="c", subcore_axis_name="s")
    run = functools.partial(
        pl.kernel,
        mesh=mesh,
        compiler_params=pltpu.CompilerParams(
            use_tc_tiling_on_sc=True, needs_layout_passes=False),
        out_type=jax.ShapeDtypeStruct((R, C), jnp.float32),
        scratch_types=[
            pltpu.VMEM((C,), jnp.float32),
            pltpu.VMEM((C,), jnp.int32),
            pltpu.VMEM((3, BIG, C), jnp.float32),
            pltpu.SemaphoreType.DMA,
            pltpu.SemaphoreType.DMA,
            pltpu.SemaphoreType.DMA,
            pltpu.SemaphoreType.DMA,
            pltpu.SemaphoreType.DMA,
            pltpu.SemaphoreType.DMA,
        ],
    )(_body)
    out = run(x, indexes)
    return out.reshape(H, W, N, C).transpose(2, 3, 0, 1)
